# baseline (speedup 1.0000x reference)
import functools

import jax
import jax.numpy as jnp
from jax import lax
from jax.experimental import pallas as pl
from jax.experimental.pallas import tpu as pltpu
from jax.experimental.pallas import tpu_sc as plsc

N_NODES = 10000
N_EDGES = 320000
D = 128
L = 16
NC = 2
NS = 16
NW = NC * NS
EPW = N_EDGES // NW
CH = 80
NCHUNK = EPW // CH
NPAD = 10112
RPT = NPAD // NS


def _node_proj_body(nf_ref, wst_ref, wdt_ref, g_ref, h_ref):
    nf = nf_ref[...]
    g_ref[...] = jnp.dot(nf, wst_ref[...], preferred_element_type=jnp.float32)
    h_ref[...] = jnp.dot(nf, wdt_ref[...], preferred_element_type=jnp.float32)


def _edge_proj_body(ef_ref, wet_ref, b_ref, em_ref):
    em_ref[...] = (
        jnp.dot(ef_ref[...], wet_ref[...], preferred_element_type=jnp.float32)
        + b_ref[...]
    )


def _update_body(nf_ref, p0_ref, p1_ref, c0_ref, c1_ref, wu1_ref, wu2_ref,
                 b_ref, out_ref):
    cnt = c0_ref[:, 0:1] + c1_ref[:, 0:1] + 1e-6
    agg = (p0_ref[...] + p1_ref[...]) / cnt
    acc = jnp.dot(nf_ref[...], wu1_ref[...], preferred_element_type=jnp.float32)
    acc += jnp.dot(agg, wu2_ref[...], preferred_element_type=jnp.float32)
    out_ref[...] = jnp.maximum(acc + b_ref[...], 0.0)


@functools.cache
def _sc_mesh():
  return plsc.VectorSubcoreMesh(
      core_axis_name="c", subcore_axis_name="s", num_cores=NC, num_subcores=NS)


@functools.cache
def _build_sc_counts():
  @functools.partial(
      pl.kernel,
      out_type=jax.ShapeDtypeStruct((NC * NPAD, D), jnp.float32),
      mesh=_sc_mesh(),
      scratch_types=[
          pltpu.VMEM((CH,), jnp.int32),
          pltpu.VMEM((CH,), jnp.int32),
          pltpu.VMEM((CH, D), jnp.float32),
          pltpu.VMEM_SHARED((NPAD, D), jnp.float32),
          pltpu.SemaphoreType.DMA,
      ],
  )
  def _sc_counts(col_hbm, cnt_out, col0, col1, ones_v, agg_sh, sc2):
    c = lax.axis_index("c")
    s = lax.axis_index("s")
    wid = s * NC + c
    zero16 = jnp.zeros((L,), jnp.float32)
    one16 = jnp.ones((L,), jnp.float32)
    base_r = s * RPT
    nfull = RPT // CH
    rem = RPT - nfull * CH
    ebase = wid * EPW
    out_base = c * NPAD + base_r
    COL = (col0, col1)

    def _fill(val):
      def _f(i, carry):
        for j in range(D // L):
          ones_v[i, pl.ds(j * L, L)] = val
        return carry
      lax.fori_loop(0, CH, _f, 0)

    _fill(zero16)
    for kk in range(nfull):
      pltpu.sync_copy(ones_v, agg_sh.at[pl.ds(base_r + kk * CH, CH)])
    if rem:
      pltpu.sync_copy(ones_v.at[pl.ds(0, rem)],
                      agg_sh.at[pl.ds(base_r + nfull * CH, rem)])
    plsc.subcore_barrier()
    _fill(one16)
    pltpu.sync_copy(col_hbm.at[pl.ds(ebase, CH)], col0)

    def _loop(i, carry):
      b0 = ebase + (2 * i) * CH
      pltpu.async_copy(ones_v, agg_sh.at[COL[0]], sc2, add=True)
      pltpu.sync_copy(col_hbm.at[pl.ds(b0 + CH, CH)], COL[1])
      pltpu.make_async_copy(ones_v, agg_sh.at[COL[0]], sc2).wait()
      pltpu.async_copy(ones_v, agg_sh.at[COL[1]], sc2, add=True)

      @pl.when(i < (NCHUNK // 2) - 1)
      def _():
        pltpu.sync_copy(col_hbm.at[pl.ds(b0 + 2 * CH, CH)], COL[0])

      pltpu.make_async_copy(ones_v, agg_sh.at[COL[1]], sc2).wait()
      return carry
    lax.fori_loop(0, NCHUNK // 2, _loop, 0)
    if NCHUNK % 2:
      pltpu.sync_copy(col_hbm.at[pl.ds(ebase + (NCHUNK - 1) * CH, CH)], col0)
      pltpu.async_copy(ones_v, agg_sh.at[COL[0]], sc2, add=True)
      pltpu.make_async_copy(ones_v, agg_sh.at[COL[0]], sc2).wait()
    plsc.subcore_barrier()
    pltpu.sync_copy(agg_sh.at[pl.ds(base_r, RPT)],
                    cnt_out.at[pl.ds(out_base, RPT)])

  return _sc_counts


@functools.cache
def _build_sc_aggregate():
  CHP = 40
  NCH = EPW // CHP

  @functools.partial(
      pl.kernel,
      out_type=jax.ShapeDtypeStruct((NC * NPAD, D), jnp.float32),
      mesh=_sc_mesh(),
      scratch_types=[
          pltpu.VMEM((CHP,), jnp.int32), pltpu.VMEM((CHP,), jnp.int32),
          pltpu.VMEM((CHP,), jnp.int32), pltpu.VMEM((CHP,), jnp.int32),
          pltpu.VMEM((CHP, D), jnp.float32), pltpu.VMEM((CHP, D), jnp.float32),
          pltpu.VMEM((CHP, D), jnp.float32), pltpu.VMEM((CHP, D), jnp.float32),
          pltpu.VMEM((CHP, D), jnp.float32), pltpu.VMEM((CHP, D), jnp.float32),
          pltpu.VMEM_SHARED((NPAD, D), jnp.float32),
          pltpu.SemaphoreType.DMA, pltpu.SemaphoreType.DMA,
          pltpu.SemaphoreType.DMA, pltpu.SemaphoreType.DMA,
          pltpu.SemaphoreType.DMA, pltpu.SemaphoreType.DMA,
          pltpu.SemaphoreType.DMA, pltpu.SemaphoreType.DMA,
      ],
  )
  def _sc_aggregate(g_hbm, h_hbm, em_hbm, row_hbm, col_hbm, agg_out,
                    row0, row1, col0, col1, g0, g1, h0, h1, e0, e1, agg_sh,
                    sg0, sg1, sh0, sh1, se0, se1, ss0, ss1):
    c = lax.axis_index("c")
    s = lax.axis_index("s")
    wid = s * NC + c
    zero16 = jnp.zeros((L,), jnp.float32)
    base_r = s * RPT
    nfullp = RPT // CHP
    remp = RPT - nfullp * CHP
    ebase = wid * EPW
    out_base = c * NPAD + base_r
    ROW = (row0, row1)
    COL = (col0, col1)
    G = (g0, g1)
    H = (h0, h1)
    E = (e0, e1)
    SG = (sg0, sg1)
    SH = (sh0, sh1)
    SE = (se0, se1)
    SS = (ss0, ss1)

    def _f(i, carry):
      for j in range(D // L):
        e0[i, pl.ds(j * L, L)] = zero16
      return carry
    lax.fori_loop(0, CHP, _f, 0)
    for kk in range(nfullp):
      pltpu.sync_copy(e0, agg_sh.at[pl.ds(base_r + kk * CHP, CHP)])
    if remp:
      pltpu.sync_copy(e0.at[pl.ds(0, remp)],
                      agg_sh.at[pl.ds(base_r + nfullp * CHP, remp)])
    plsc.subcore_barrier()

    def _issue(b, p):
      pltpu.sync_copy(row_hbm.at[pl.ds(b, CHP)], ROW[p])
      pltpu.sync_copy(col_hbm.at[pl.ds(b, CHP)], COL[p])
      pltpu.async_copy(em_hbm.at[pl.ds(b, CHP)], E[p], SE[p])
      pltpu.async_copy(g_hbm.at[ROW[p]], G[p], SG[p])
      pltpu.async_copy(h_hbm.at[COL[p]], H[p], SH[p])

    def _wait_inputs(b, p):
      pltpu.make_async_copy(em_hbm.at[pl.ds(b, CHP)], E[p], SE[p]).wait()
      pltpu.make_async_copy(g_hbm.at[ROW[p]], G[p], SG[p]).wait()
      pltpu.make_async_copy(h_hbm.at[COL[p]], H[p], SH[p]).wait()

    def _compute(p):
      def _row(i, c2):
        for j in range(D // L):
          sl = pl.ds(j * L, L)
          v = G[p][i, sl] + H[p][i, sl] + E[p][i, sl]
          E[p][i, sl] = jnp.maximum(v, 0.0)
        return c2
      lax.fori_loop(0, CHP, _row, 0)

    def _scatter(p):
      pltpu.async_copy(E[p], agg_sh.at[COL[p]], SS[p], add=True)

    def _drain(p):
      pltpu.make_async_copy(E[p], agg_sh.at[COL[p]], SS[p]).wait()

    _issue(ebase, 0)
    _wait_inputs(ebase, 0)
    _compute(0)
    _scatter(0)
    _issue(ebase + CHP, 1)
    _wait_inputs(ebase + CHP, 1)
    _compute(1)
    _scatter(1)
    _drain(0)
    _issue(ebase + 2 * CHP, 0)

    def _loop(i, carry):
      b0 = ebase + (2 * i + 2) * CHP
      _wait_inputs(b0, 0)
      _compute(0)
      _scatter(0)
      _drain(1)
      _issue(b0 + CHP, 1)
      b1 = b0 + CHP
      _wait_inputs(b1, 1)
      _compute(1)
      _scatter(1)
      _drain(0)
      _issue(b1 + CHP, 0)
      return carry
    lax.fori_loop(0, (NCH - 4) // 2, _loop, 0)
    b_pen = ebase + (NCH - 2) * CHP
    _wait_inputs(b_pen, 0)
    _compute(0)
    _scatter(0)
    _drain(1)
    _issue(b_pen + CHP, 1)
    _wait_inputs(b_pen + CHP, 1)
    _compute(1)
    _scatter(1)
    _drain(0)
    _drain(1)
    plsc.subcore_barrier()
    pltpu.sync_copy(agg_sh.at[pl.ds(base_r, RPT)],
                    agg_out.at[pl.ds(out_base, RPT)])

  return _sc_aggregate


def kernel(node_features, edge_features, edge_index, W_msg, b_msg, W_upd, b_upd):
    nf = node_features.astype(jnp.float32)
    ef = edge_features.astype(jnp.float32)
    row = edge_index[0].astype(jnp.int32)
    col = edge_index[1].astype(jnp.int32)

    wst = W_msg[:, 0:D].T
    wdt = W_msg[:, D:2 * D].T
    wet = W_msg[:, 2 * D:3 * D].T
    wu1 = W_upd[:, 0:D].T
    wu2 = W_upd[:, D:2 * D].T
    bm = b_msg.reshape(1, D)
    bu = b_upd.reshape(1, D)

    cnt = _build_sc_counts()(col)

    nb = 2000
    g, h = pl.pallas_call(
        _node_proj_body,
        grid=(N_NODES // nb,),
        in_specs=[
            pl.BlockSpec((nb, D), lambda i: (i, 0)),
            pl.BlockSpec((D, D), lambda i: (0, 0)),
            pl.BlockSpec((D, D), lambda i: (0, 0)),
        ],
        out_specs=[
            pl.BlockSpec((nb, D), lambda i: (i, 0)),
            pl.BlockSpec((nb, D), lambda i: (i, 0)),
        ],
        out_shape=[
            jax.ShapeDtypeStruct((N_NODES, D), jnp.float32),
            jax.ShapeDtypeStruct((N_NODES, D), jnp.float32),
        ],
    )(nf, wst, wdt)

    eb = 2000
    em = pl.pallas_call(
        _edge_proj_body,
        grid=(N_EDGES // eb,),
        in_specs=[
            pl.BlockSpec((eb, D), lambda i: (i, 0)),
            pl.BlockSpec((D, D), lambda i: (0, 0)),
            pl.BlockSpec((1, D), lambda i: (0, 0)),
        ],
        out_specs=pl.BlockSpec((eb, D), lambda i: (i, 0)),
        out_shape=jax.ShapeDtypeStruct((N_EDGES, D), jnp.float32),
    )(ef, wet, bm)

    agg = _build_sc_aggregate()(g, h, em, row, col)

    p0 = lax.slice(agg, (0, 0), (N_NODES, D))
    p1 = lax.slice(agg, (NPAD, 0), (NPAD + N_NODES, D))
    c0 = lax.slice(cnt, (0, 0), (N_NODES, L))
    c1 = lax.slice(cnt, (NPAD, 0), (NPAD + N_NODES, L))

    out = pl.pallas_call(
        _update_body,
        grid=(N_NODES // nb,),
        in_specs=[
            pl.BlockSpec((nb, D), lambda i: (i, 0)),
            pl.BlockSpec((nb, D), lambda i: (i, 0)),
            pl.BlockSpec((nb, D), lambda i: (i, 0)),
            pl.BlockSpec((nb, L), lambda i: (i, 0)),
            pl.BlockSpec((nb, L), lambda i: (i, 0)),
            pl.BlockSpec((D, D), lambda i: (0, 0)),
            pl.BlockSpec((D, D), lambda i: (0, 0)),
            pl.BlockSpec((1, D), lambda i: (0, 0)),
        ],
        out_specs=pl.BlockSpec((nb, D), lambda i: (i, 0)),
        out_shape=jax.ShapeDtypeStruct((N_NODES, D), jnp.float32),
    )(nf, p0, p1, c0, c1, wu1, wu2, bu)

    return out


# speedup vs baseline: 1.2026x; 1.2026x over previous
import functools

import jax
import jax.numpy as jnp
from jax import lax
from jax.experimental import pallas as pl
from jax.experimental.pallas import tpu as pltpu
from jax.experimental.pallas import tpu_sc as plsc

N_NODES = 10000
N_EDGES = 320000
D = 128
L = 16
NC = 2
NS = 16
NW = NC * NS
EPW = N_EDGES // NW
CH = 80
NCHUNK = EPW // CH
NPAD = 10112
RPT = NPAD // NS


def _node_proj_body(nf_ref, wst_ref, wdt_ref, g_ref, h_ref):
    nf = nf_ref[...]
    g_ref[...] = jnp.dot(nf, wst_ref[...], preferred_element_type=jnp.float32)
    h_ref[...] = jnp.dot(nf, wdt_ref[...], preferred_element_type=jnp.float32)


def _edge_proj_body(ef_ref, wet_ref, b_ref, em_ref):
    em_ref[...] = (
        jnp.dot(ef_ref[...], wet_ref[...], preferred_element_type=jnp.float32)
        + b_ref[...]
    )


def _update_body(nf_ref, p0_ref, p1_ref, c0_ref, c1_ref, wu1_ref, wu2_ref,
                 b_ref, out_ref):
    cnt = c0_ref[:, 0:1] + c1_ref[:, 0:1] + 1e-6
    agg = (p0_ref[...] + p1_ref[...]) / cnt
    acc = jnp.dot(nf_ref[...], wu1_ref[...], preferred_element_type=jnp.float32)
    acc += jnp.dot(agg, wu2_ref[...], preferred_element_type=jnp.float32)
    out_ref[...] = jnp.maximum(acc + b_ref[...], 0.0)


@functools.cache
def _sc_mesh():
  return plsc.VectorSubcoreMesh(
      core_axis_name="c", subcore_axis_name="s", num_cores=NC, num_subcores=NS)


@functools.cache
def _build_sc_counts():
  @functools.partial(
      pl.kernel,
      out_type=jax.ShapeDtypeStruct((NC * NPAD, D), jnp.float32),
      mesh=_sc_mesh(),
      scratch_types=[
          pltpu.VMEM((CH,), jnp.int32),
          pltpu.VMEM((CH,), jnp.int32),
          pltpu.VMEM((CH, D), jnp.float32),
          pltpu.VMEM_SHARED((NPAD, D), jnp.float32),
          pltpu.SemaphoreType.DMA,
      ],
  )
  def _sc_counts(col_hbm, cnt_out, col0, col1, ones_v, agg_sh, sc2):
    c = lax.axis_index("c")
    s = lax.axis_index("s")
    wid = s * NC + c
    zero16 = jnp.zeros((L,), jnp.float32)
    one16 = jnp.ones((L,), jnp.float32)
    base_r = s * RPT
    nfull = RPT // CH
    rem = RPT - nfull * CH
    ebase = wid * EPW
    out_base = c * NPAD + base_r
    COL = (col0, col1)

    def _fill(val):
      def _f(i, carry):
        for j in range(D // L):
          ones_v[i, pl.ds(j * L, L)] = val
        return carry
      lax.fori_loop(0, CH, _f, 0)

    _fill(zero16)
    for kk in range(nfull):
      pltpu.sync_copy(ones_v, agg_sh.at[pl.ds(base_r + kk * CH, CH)])
    if rem:
      pltpu.sync_copy(ones_v.at[pl.ds(0, rem)],
                      agg_sh.at[pl.ds(base_r + nfull * CH, rem)])
    plsc.subcore_barrier()
    _fill(one16)
    pltpu.sync_copy(col_hbm.at[pl.ds(ebase, CH)], col0)

    def _loop(i, carry):
      b0 = ebase + (2 * i) * CH
      pltpu.async_copy(ones_v, agg_sh.at[COL[0]], sc2, add=True)
      pltpu.sync_copy(col_hbm.at[pl.ds(b0 + CH, CH)], COL[1])
      pltpu.make_async_copy(ones_v, agg_sh.at[COL[0]], sc2).wait()
      pltpu.async_copy(ones_v, agg_sh.at[COL[1]], sc2, add=True)

      @pl.when(i < (NCHUNK // 2) - 1)
      def _():
        pltpu.sync_copy(col_hbm.at[pl.ds(b0 + 2 * CH, CH)], COL[0])

      pltpu.make_async_copy(ones_v, agg_sh.at[COL[1]], sc2).wait()
      return carry
    lax.fori_loop(0, NCHUNK // 2, _loop, 0)
    if NCHUNK % 2:
      pltpu.sync_copy(col_hbm.at[pl.ds(ebase + (NCHUNK - 1) * CH, CH)], col0)
      pltpu.async_copy(ones_v, agg_sh.at[COL[0]], sc2, add=True)
      pltpu.make_async_copy(ones_v, agg_sh.at[COL[0]], sc2).wait()
    plsc.subcore_barrier()
    pltpu.sync_copy(agg_sh.at[pl.ds(base_r, RPT)],
                    cnt_out.at[pl.ds(out_base, RPT)])

  return _sc_counts


@functools.cache
def _build_sc_aggregate():
  @functools.partial(
      pl.kernel,
      out_type=jax.ShapeDtypeStruct((NC * NPAD, D), jnp.float32),
      mesh=_sc_mesh(),
      scratch_types=[
          pltpu.VMEM((CH,), jnp.int32),
          pltpu.VMEM((CH,), jnp.int32),
          pltpu.VMEM((CH,), jnp.int32),
          pltpu.VMEM((CH, D), jnp.float32),
          pltpu.VMEM((CH, D), jnp.float32),
          pltpu.VMEM((CH, D), jnp.float32),
          pltpu.VMEM((CH, D), jnp.float32),
          pltpu.VMEM_SHARED((NPAD, D), jnp.float32),
          pltpu.SemaphoreType.DMA,
          pltpu.SemaphoreType.DMA,
          pltpu.SemaphoreType.DMA,
          pltpu.SemaphoreType.DMA,
          pltpu.SemaphoreType.DMA,
      ],
  )
  def _sc_aggregate(g_hbm, h_hbm, em_hbm, row_hbm, col_hbm, agg_out,
                    row_v, col0, col1, g_v, h_v, e0, e1, agg_sh,
                    sem_g, sem_h, se0, se1, ss):
    c = lax.axis_index("c")
    s = lax.axis_index("s")
    wid = s * NC + c
    zero16 = jnp.zeros((L,), jnp.float32)
    base_r = s * RPT
    nfull = RPT // CH
    rem = RPT - nfull * CH
    ebase = wid * EPW
    out_base = c * NPAD + base_r

    COL = (col0, col1)
    E = (e0, e1)
    SE = (se0, se1)

    def _f(i, carry):
      for j in range(D // L):
        e0[i, pl.ds(j * L, L)] = zero16
      return carry
    lax.fori_loop(0, CH, _f, 0)
    for kk in range(nfull):
      pltpu.sync_copy(e0, agg_sh.at[pl.ds(base_r + kk * CH, CH)])
    if rem:
      pltpu.sync_copy(e0.at[pl.ds(0, rem)],
                      agg_sh.at[pl.ds(base_r + nfull * CH, rem)])
    plsc.subcore_barrier()

    def _issue(b, p):
      pltpu.sync_copy(row_hbm.at[pl.ds(b, CH)], row_v)
      pltpu.sync_copy(col_hbm.at[pl.ds(b, CH)], COL[p])
      pltpu.async_copy(em_hbm.at[pl.ds(b, CH)], E[p], SE[p])
      pltpu.async_copy(g_hbm.at[row_v], g_v, sem_g)
      pltpu.async_copy(h_hbm.at[COL[p]], h_v, sem_h)

    def _wait_inputs(b, p):
      pltpu.make_async_copy(em_hbm.at[pl.ds(b, CH)], E[p], SE[p]).wait()
      pltpu.make_async_copy(g_hbm.at[row_v], g_v, sem_g).wait()
      pltpu.make_async_copy(h_hbm.at[COL[p]], h_v, sem_h).wait()

    def _compute(p):
      def _row(i, c2):
        for j in range(D // L):
          sl = pl.ds(j * L, L)
          v = g_v[i, sl] + h_v[i, sl] + E[p][i, sl]
          E[p][i, sl] = jnp.maximum(v, 0.0)
        return c2
      lax.fori_loop(0, CH, _row, 0)

    _issue(ebase, 0)
    _wait_inputs(ebase, 0)
    _compute(0)
    pltpu.async_copy(E[0], agg_sh.at[COL[0]], ss, add=True)
    _issue(ebase + CH, 1)

    def _chunk(k2, carry):
      b0 = ebase + (2 * k2 + 1) * CH
      _wait_inputs(b0, 1)
      pltpu.make_async_copy(E[0], agg_sh.at[COL[0]], ss).wait()
      _compute(1)
      pltpu.async_copy(E[1], agg_sh.at[COL[1]], ss, add=True)
      _issue(b0 + CH, 0)
      b1 = b0 + CH
      _wait_inputs(b1, 0)
      pltpu.make_async_copy(E[1], agg_sh.at[COL[1]], ss).wait()
      _compute(0)
      pltpu.async_copy(E[0], agg_sh.at[COL[0]], ss, add=True)
      _issue(b1 + CH, 1)
      return carry
    lax.fori_loop(0, (NCHUNK - 2) // 2, _chunk, 0)
    b_pen = ebase + (NCHUNK - 2) * CH
    _wait_inputs(b_pen, 1)
    pltpu.make_async_copy(E[0], agg_sh.at[COL[0]], ss).wait()
    _compute(1)
    pltpu.async_copy(E[1], agg_sh.at[COL[1]], ss, add=True)
    _issue(b_pen + CH, 0)
    _wait_inputs(b_pen + CH, 0)
    pltpu.make_async_copy(E[1], agg_sh.at[COL[1]], ss).wait()
    _compute(0)
    pltpu.async_copy(E[0], agg_sh.at[COL[0]], ss, add=True)
    pltpu.make_async_copy(E[0], agg_sh.at[COL[0]], ss).wait()
    plsc.subcore_barrier()
    pltpu.sync_copy(agg_sh.at[pl.ds(base_r, RPT)],
                    agg_out.at[pl.ds(out_base, RPT)])

  return _sc_aggregate


def kernel(node_features, edge_features, edge_index, W_msg, b_msg, W_upd, b_upd):
    nf = node_features.astype(jnp.float32)
    ef = edge_features.astype(jnp.float32)
    row = edge_index[0].astype(jnp.int32)
    col = edge_index[1].astype(jnp.int32)

    wst = W_msg[:, 0:D].T
    wdt = W_msg[:, D:2 * D].T
    wet = W_msg[:, 2 * D:3 * D].T
    wu1 = W_upd[:, 0:D].T
    wu2 = W_upd[:, D:2 * D].T
    bm = b_msg.reshape(1, D)
    bu = b_upd.reshape(1, D)

    cnt = _build_sc_counts()(col)

    nb = 2000
    g, h = pl.pallas_call(
        _node_proj_body,
        grid=(N_NODES // nb,),
        in_specs=[
            pl.BlockSpec((nb, D), lambda i: (i, 0)),
            pl.BlockSpec((D, D), lambda i: (0, 0)),
            pl.BlockSpec((D, D), lambda i: (0, 0)),
        ],
        out_specs=[
            pl.BlockSpec((nb, D), lambda i: (i, 0)),
            pl.BlockSpec((nb, D), lambda i: (i, 0)),
        ],
        out_shape=[
            jax.ShapeDtypeStruct((N_NODES, D), jnp.float32),
            jax.ShapeDtypeStruct((N_NODES, D), jnp.float32),
        ],
    )(nf, wst, wdt)

    eb = 2000
    em = pl.pallas_call(
        _edge_proj_body,
        grid=(N_EDGES // eb,),
        in_specs=[
            pl.BlockSpec((eb, D), lambda i: (i, 0)),
            pl.BlockSpec((D, D), lambda i: (0, 0)),
            pl.BlockSpec((1, D), lambda i: (0, 0)),
        ],
        out_specs=pl.BlockSpec((eb, D), lambda i: (i, 0)),
        out_shape=jax.ShapeDtypeStruct((N_EDGES, D), jnp.float32),
    )(ef, wet, bm)

    agg = _build_sc_aggregate()(g, h, em, row, col)

    p0 = lax.slice(agg, (0, 0), (N_NODES, D))
    p1 = lax.slice(agg, (NPAD, 0), (NPAD + N_NODES, D))
    c0 = lax.slice(cnt, (0, 0), (N_NODES, L))
    c1 = lax.slice(cnt, (NPAD, 0), (NPAD + N_NODES, L))

    out = pl.pallas_call(
        _update_body,
        grid=(N_NODES // nb,),
        in_specs=[
            pl.BlockSpec((nb, D), lambda i: (i, 0)),
            pl.BlockSpec((nb, D), lambda i: (i, 0)),
            pl.BlockSpec((nb, D), lambda i: (i, 0)),
            pl.BlockSpec((nb, L), lambda i: (i, 0)),
            pl.BlockSpec((nb, L), lambda i: (i, 0)),
            pl.BlockSpec((D, D), lambda i: (0, 0)),
            pl.BlockSpec((D, D), lambda i: (0, 0)),
            pl.BlockSpec((1, D), lambda i: (0, 0)),
        ],
        out_specs=pl.BlockSpec((nb, D), lambda i: (i, 0)),
        out_shape=jax.ShapeDtypeStruct((N_NODES, D), jnp.float32),
    )(nf, p0, p1, c0, c1, wu1, wu2, bu)

    return out
